# baseline (speedup 1.0000x reference)
import functools

import jax
import jax.numpy as jnp
from jax import lax
from jax.experimental import pallas as pl
from jax.experimental.pallas import tpu as pltpu
from jax.experimental.pallas import tpu_sc as plsc

NUM_CODES = 8192
DIM = 256
NUM_TOKENS = 8192
TM = 256
TN = 2048
N_TOK_TILES = NUM_TOKENS // TM
N_CODE_TILES = NUM_CODES // TN


def _argmin_body(x_ref, e_ref, out_ref, dbuf_ref, esq_ref, xsq_ref):
    s = pl.program_id(0)

    @pl.when(s == 0)
    def _():
        e0 = e_ref[...]
        esq_ref[...] = jnp.sum(e0 * e0, axis=1)[None, :]

    p_w = lax.rem(s, 2)
    p_r = 1 - p_w

    x = x_ref[...]
    xsq = jnp.sum(x * x, axis=1, keepdims=True)
    dot2 = lax.dot_general(x * jnp.float32(-2.0), e_ref[...],
                           (((1,), (1,)), ((), ())),
                           preferred_element_type=jnp.float32)
    dbuf_ref[pl.ds(p_w, 1)] = dot2[None]
    xsq_ref[pl.ds(p_w, 1)] = xsq[None]

    dot2p = dbuf_ref[pl.ds(p_r, 1)][0]
    dist = (xsq_ref[pl.ds(p_r, 1)][0] + dot2p) + esq_ref[...]
    m = jnp.min(dist, axis=1, keepdims=True)
    iota = lax.broadcasted_iota(
        jnp.int32, (1, NUM_CODES), 1).astype(jnp.float32)
    cand = jnp.where(dist == m, iota, jnp.float32(1e9))
    idx = jnp.min(cand, axis=1, keepdims=True)
    out_ref[0, :, :] = idx.astype(jnp.int32)


def _argmin_call(flat, emb):
    out = pl.pallas_call(
        _argmin_body,
        grid=(N_TOK_TILES + 1,),
        in_specs=[
            pl.BlockSpec((TM, DIM),
                         lambda s: (jnp.minimum(s, N_TOK_TILES - 1), 0)),
            pl.BlockSpec((NUM_CODES, DIM), lambda s: (0, 0)),
        ],
        out_specs=pl.BlockSpec((1, TM, 1),
                               lambda s: (jnp.maximum(s, 1) - 1, 0, 0)),
        out_shape=jax.ShapeDtypeStruct((N_TOK_TILES, TM, 1), jnp.int32),
        scratch_shapes=[
            pltpu.VMEM((2, TM, NUM_CODES), jnp.float32),
            pltpu.VMEM((1, NUM_CODES), jnp.float32),
            pltpu.VMEM((2, TM, 1), jnp.float32),
        ],
    )(flat, emb)
    return out.reshape(NUM_TOKENS)


_NC = 2
_NS = 16
_NW = _NC * _NS
_CHUNK = 128
_ROWS = NUM_TOKENS // _CHUNK
_RPW = _ROWS // _NW


def _sc_gather_body(table_hbm, idx_hbm, out_hbm, idx_v, rows_v, sem):
    wid = lax.axis_index("s") * _NC + lax.axis_index("c")
    r0 = wid * _RPW
    pltpu.sync_copy(idx_hbm.at[pl.ds(r0, _RPW)], idx_v)
    cps = [
        pltpu.async_copy(table_hbm.at[idx_v.at[r]], rows_v.at[r], sem)
        for r in range(_RPW)
    ]
    for cp in cps:
        cp.wait()
    pltpu.sync_copy(rows_v, out_hbm.at[pl.ds(r0, _RPW)])


@functools.cache
def _sc_gather():
    return pl.kernel(
        _sc_gather_body,
        mesh=plsc.VectorSubcoreMesh(core_axis_name="c", subcore_axis_name="s"),
        out_type=jax.ShapeDtypeStruct((_ROWS, _CHUNK, DIM), jnp.float32),
        scratch_types=[
            pltpu.VMEM((_RPW, _CHUNK), jnp.int32),
            pltpu.VMEM((_RPW, _CHUNK, DIM), jnp.float32),
            pltpu.SemaphoreType.DMA,
        ],
    )


FT = 1024
N_FIN = NUM_TOKENS // FT


def _finalize_body(x_ref, q_ref, qst_ref, loss_ref, acc_ref):
    i = pl.program_id(0)
    x = x_ref[...]
    q = q_ref[...]
    d = q - x
    qst_ref[...] = x + d
    s = jnp.sum(d * d)

    @pl.when(i == 0)
    def _():
        acc_ref[0, 0] = s

    @pl.when(i > 0)
    def _():
        acc_ref[0, 0] = acc_ref[0, 0] + s

    @pl.when(i == pl.num_programs(0) - 1)
    def _():
        m = acc_ref[0, 0] / jnp.float32(NUM_TOKENS * DIM)
        loss_ref[0, 0] = m + 0.25 * m


def _finalize_call(flat, q):
    return pl.pallas_call(
        _finalize_body,
        grid=(N_FIN,),
        in_specs=[
            pl.BlockSpec((FT, DIM), lambda i: (i, 0)),
            pl.BlockSpec((FT, DIM), lambda i: (i, 0)),
        ],
        out_specs=[
            pl.BlockSpec((FT, DIM), lambda i: (i, 0)),
            pl.BlockSpec(memory_space=pltpu.SMEM),
        ],
        out_shape=[
            jax.ShapeDtypeStruct((NUM_TOKENS, DIM), jnp.float32),
            jax.ShapeDtypeStruct((1, 1), jnp.float32),
        ],
        scratch_shapes=[pltpu.SMEM((1, 1), jnp.float32)],
    )(flat, q)


def kernel(inputs, embedding):
    B, C, H, W = inputs.shape
    flat = jnp.transpose(inputs, (0, 2, 3, 1)).reshape(NUM_TOKENS, DIM)
    idx = _argmin_call(flat, embedding)
    q = _sc_gather()(embedding, idx.reshape(_ROWS, _CHUNK)).reshape(NUM_TOKENS, DIM)
    qst_flat, loss = _finalize_call(flat, q)
    qst = jnp.transpose(qst_flat.reshape(B, H, W, C), (0, 3, 1, 2))
    return qst, loss[0, 0], idx.reshape(B, H, W)


# speedup vs baseline: 1.2536x; 1.2536x over previous
import functools

import jax
import jax.numpy as jnp
from jax import lax
from jax.experimental import pallas as pl
from jax.experimental.pallas import tpu as pltpu
from jax.experimental.pallas import tpu_sc as plsc

NUM_CODES = 8192
DIM = 256
NUM_TOKENS = 8192
TM = 256
TN = 2048
N_TOK_TILES = NUM_TOKENS // TM
N_CODE_TILES = NUM_CODES // TN


def _argmin_body(x_ref, e_ref, out_ref, dbuf_ref, esq_ref, xsq_ref):
    s = pl.program_id(0)

    @pl.when(s == 0)
    def _():
        e0 = e_ref[...]
        esq_ref[...] = jnp.sum(e0 * e0, axis=1)[None, :]

    @pl.when(s < N_TOK_TILES)
    def _():
        x = x_ref[...]
        xsq = jnp.sum(x * x, axis=1, keepdims=True)
        dot2 = lax.dot_general(x * jnp.float32(-2.0), e_ref[...],
                               (((1,), (1,)), ((), ())),
                               preferred_element_type=jnp.float32)

        @pl.when(s % 2 == 0)
        def _():
            dbuf_ref[0] = dot2
            xsq_ref[0] = xsq

        @pl.when(s % 2 == 1)
        def _():
            dbuf_ref[1] = dot2
            xsq_ref[1] = xsq

    @pl.when(s > 0)
    def _():
        def epilogue(parity):
            dot2 = dbuf_ref[parity]
            dist = (xsq_ref[parity] + dot2) + esq_ref[...]
            m = jnp.min(dist, axis=1, keepdims=True)
            iota = lax.broadcasted_iota(
                jnp.int32, (1, NUM_CODES), 1).astype(jnp.float32)
            cand = jnp.where(dist == m, iota, jnp.float32(1e9))
            idx = jnp.min(cand, axis=1, keepdims=True)
            out_ref[0, :, :] = idx.astype(jnp.int32)

        @pl.when(s % 2 == 1)
        def _():
            epilogue(0)

        @pl.when(s % 2 == 0)
        def _():
            epilogue(1)


def _argmin_call(flat, emb):
    out = pl.pallas_call(
        _argmin_body,
        grid=(N_TOK_TILES + 1,),
        in_specs=[
            pl.BlockSpec((TM, DIM),
                         lambda s: (jnp.minimum(s, N_TOK_TILES - 1), 0)),
            pl.BlockSpec((NUM_CODES, DIM), lambda s: (0, 0)),
        ],
        out_specs=pl.BlockSpec((1, TM, 1),
                               lambda s: (jnp.maximum(s, 1) - 1, 0, 0)),
        out_shape=jax.ShapeDtypeStruct((N_TOK_TILES, TM, 1), jnp.int32),
        scratch_shapes=[
            pltpu.VMEM((2, TM, NUM_CODES), jnp.float32),
            pltpu.VMEM((1, NUM_CODES), jnp.float32),
            pltpu.VMEM((2, TM, 1), jnp.float32),
        ],
    )(flat, emb)
    return out.reshape(NUM_TOKENS)


_NC = 2
_NS = 16
_NW = _NC * _NS
_CHUNK = 128
_ROWS = NUM_TOKENS // _CHUNK
_RPW = _ROWS // _NW


def _sc_gather_body(table_hbm, idx_hbm, out_hbm, idx_v, rows_v, sem):
    wid = lax.axis_index("s") * _NC + lax.axis_index("c")
    r0 = wid * _RPW
    pltpu.sync_copy(idx_hbm.at[pl.ds(r0, _RPW)], idx_v)
    cps = [
        pltpu.async_copy(table_hbm.at[idx_v.at[r]], rows_v.at[r], sem)
        for r in range(_RPW)
    ]
    for cp in cps:
        cp.wait()
    pltpu.sync_copy(rows_v, out_hbm.at[pl.ds(r0, _RPW)])


@functools.cache
def _sc_gather():
    return pl.kernel(
        _sc_gather_body,
        mesh=plsc.VectorSubcoreMesh(core_axis_name="c", subcore_axis_name="s"),
        out_type=jax.ShapeDtypeStruct((_ROWS, _CHUNK, DIM), jnp.float32),
        scratch_types=[
            pltpu.VMEM((_RPW, _CHUNK), jnp.int32),
            pltpu.VMEM((_RPW, _CHUNK, DIM), jnp.float32),
            pltpu.SemaphoreType.DMA,
        ],
    )


FT = 1024
N_FIN = NUM_TOKENS // FT


def _finalize_body(x_ref, q_ref, qst_ref, loss_ref, acc_ref):
    i = pl.program_id(0)
    x = x_ref[...]
    q = q_ref[...]
    d = q - x
    qst_ref[...] = x + d
    s = jnp.sum(d * d)

    @pl.when(i == 0)
    def _():
        acc_ref[0, 0] = s

    @pl.when(i > 0)
    def _():
        acc_ref[0, 0] = acc_ref[0, 0] + s

    @pl.when(i == pl.num_programs(0) - 1)
    def _():
        m = acc_ref[0, 0] / jnp.float32(NUM_TOKENS * DIM)
        loss_ref[0, 0] = m + 0.25 * m


def _finalize_call(flat, q):
    return pl.pallas_call(
        _finalize_body,
        grid=(N_FIN,),
        in_specs=[
            pl.BlockSpec((FT, DIM), lambda i: (i, 0)),
            pl.BlockSpec((FT, DIM), lambda i: (i, 0)),
        ],
        out_specs=[
            pl.BlockSpec((FT, DIM), lambda i: (i, 0)),
            pl.BlockSpec(memory_space=pltpu.SMEM),
        ],
        out_shape=[
            jax.ShapeDtypeStruct((NUM_TOKENS, DIM), jnp.float32),
            jax.ShapeDtypeStruct((1, 1), jnp.float32),
        ],
        scratch_shapes=[pltpu.SMEM((1, 1), jnp.float32)],
    )(flat, q)


def kernel(inputs, embedding):
    B, C, H, W = inputs.shape
    flat = jnp.transpose(inputs, (0, 2, 3, 1)).reshape(NUM_TOKENS, DIM)
    idx = _argmin_call(flat, embedding)
    q = _sc_gather()(embedding, idx.reshape(_ROWS, _CHUNK)).reshape(NUM_TOKENS, DIM)
    qst_flat, loss = _finalize_call(flat, q)
    qst = jnp.transpose(qst_flat.reshape(B, H, W, C), (0, 3, 1, 2))
    return qst, loss[0, 0], idx.reshape(B, H, W)


# speedup vs baseline: 1.3409x; 1.0697x over previous
import functools

import jax
import jax.numpy as jnp
from jax import lax
from jax.experimental import pallas as pl
from jax.experimental.pallas import tpu as pltpu
from jax.experimental.pallas import tpu_sc as plsc

NUM_CODES = 8192
DIM = 256
NUM_TOKENS = 8192
TM = 256
TN = 2048
N_TOK_TILES = NUM_TOKENS // TM
N_CODE_TILES = NUM_CODES // TN


def _argmin_body(x_ref, e_ref, out_ref, dbuf_ref, esq_ref, xsq_ref):
    s = pl.program_id(0)

    @pl.when(s == 0)
    def _():
        e0 = e_ref[...]
        esq_ref[...] = jnp.sum(e0 * e0, axis=1)[None, :]

    def step_body(wbuf, rbuf):
        x = x_ref[...]
        xsq = jnp.sum(x * x, axis=1, keepdims=True)
        dot2 = lax.dot_general(x * jnp.float32(-2.0), e_ref[...],
                               (((1,), (1,)), ((), ())),
                               preferred_element_type=jnp.float32)
        dbuf_ref[wbuf] = dot2
        xsq_ref[wbuf] = xsq

        dot2p = dbuf_ref[rbuf]
        dist = (xsq_ref[rbuf] + dot2p) + esq_ref[...]
        m = jnp.min(dist, axis=1, keepdims=True)
        iota = lax.broadcasted_iota(
            jnp.int32, (1, NUM_CODES), 1).astype(jnp.float32)
        cand = jnp.where(dist == m, iota, jnp.float32(1e9))
        idx = jnp.min(cand, axis=1, keepdims=True)
        out_ref[0, :, :] = idx.astype(jnp.int32)

    @pl.when(s % 2 == 0)
    def _():
        step_body(0, 1)

    @pl.when(s % 2 == 1)
    def _():
        step_body(1, 0)


def _argmin_call(flat, emb):
    out = pl.pallas_call(
        _argmin_body,
        grid=(N_TOK_TILES + 1,),
        in_specs=[
            pl.BlockSpec((TM, DIM),
                         lambda s: (jnp.minimum(s, N_TOK_TILES - 1), 0)),
            pl.BlockSpec((NUM_CODES, DIM), lambda s: (0, 0)),
        ],
        out_specs=pl.BlockSpec((1, TM, 1),
                               lambda s: (jnp.maximum(s, 1) - 1, 0, 0)),
        out_shape=jax.ShapeDtypeStruct((N_TOK_TILES, TM, 1), jnp.int32),
        scratch_shapes=[
            pltpu.VMEM((2, TM, NUM_CODES), jnp.float32),
            pltpu.VMEM((1, NUM_CODES), jnp.float32),
            pltpu.VMEM((2, TM, 1), jnp.float32),
        ],
    )(flat, emb)
    return out.reshape(NUM_TOKENS)


_NC = 2
_NS = 16
_NW = _NC * _NS
_CHUNK = 128
_ROWS = NUM_TOKENS // _CHUNK
_RPW = _ROWS // _NW


def _sc_gather_body(table_hbm, idx_hbm, out_hbm, idx_v, rows_v, sem):
    wid = lax.axis_index("s") * _NC + lax.axis_index("c")
    r0 = wid * _RPW
    pltpu.sync_copy(idx_hbm.at[pl.ds(r0, _RPW)], idx_v)
    cps = [
        pltpu.async_copy(table_hbm.at[idx_v.at[r]], rows_v.at[r], sem)
        for r in range(_RPW)
    ]
    for cp in cps:
        cp.wait()
    pltpu.sync_copy(rows_v, out_hbm.at[pl.ds(r0, _RPW)])


@functools.cache
def _sc_gather():
    return pl.kernel(
        _sc_gather_body,
        mesh=plsc.VectorSubcoreMesh(core_axis_name="c", subcore_axis_name="s"),
        out_type=jax.ShapeDtypeStruct((_ROWS, _CHUNK, DIM), jnp.float32),
        scratch_types=[
            pltpu.VMEM((_RPW, _CHUNK), jnp.int32),
            pltpu.VMEM((_RPW, _CHUNK, DIM), jnp.float32),
            pltpu.SemaphoreType.DMA,
        ],
    )


FT = 1024
N_FIN = NUM_TOKENS // FT


def _finalize_body(x_ref, q_ref, qst_ref, loss_ref, acc_ref):
    i = pl.program_id(0)
    x = x_ref[...]
    q = q_ref[...]
    d = q - x
    qst_ref[...] = x + d
    s = jnp.sum(d * d)

    @pl.when(i == 0)
    def _():
        acc_ref[0, 0] = s

    @pl.when(i > 0)
    def _():
        acc_ref[0, 0] = acc_ref[0, 0] + s

    @pl.when(i == pl.num_programs(0) - 1)
    def _():
        m = acc_ref[0, 0] / jnp.float32(NUM_TOKENS * DIM)
        loss_ref[0, 0] = m + 0.25 * m


def _finalize_call(flat, q):
    return pl.pallas_call(
        _finalize_body,
        grid=(N_FIN,),
        in_specs=[
            pl.BlockSpec((FT, DIM), lambda i: (i, 0)),
            pl.BlockSpec((FT, DIM), lambda i: (i, 0)),
        ],
        out_specs=[
            pl.BlockSpec((FT, DIM), lambda i: (i, 0)),
            pl.BlockSpec(memory_space=pltpu.SMEM),
        ],
        out_shape=[
            jax.ShapeDtypeStruct((NUM_TOKENS, DIM), jnp.float32),
            jax.ShapeDtypeStruct((1, 1), jnp.float32),
        ],
        scratch_shapes=[pltpu.SMEM((1, 1), jnp.float32)],
    )(flat, q)


def kernel(inputs, embedding):
    B, C, H, W = inputs.shape
    flat = jnp.transpose(inputs, (0, 2, 3, 1)).reshape(NUM_TOKENS, DIM)
    idx = _argmin_call(flat, embedding)
    q = _sc_gather()(embedding, idx.reshape(_ROWS, _CHUNK)).reshape(NUM_TOKENS, DIM)
    qst_flat, loss = _finalize_call(flat, q)
    qst = jnp.transpose(qst_flat.reshape(B, H, W, C), (0, 3, 1, 2))
    return qst, loss[0, 0], idx.reshape(B, H, W)


# speedup vs baseline: 1.3721x; 1.0232x over previous
import functools

import jax
import jax.numpy as jnp
from jax import lax
from jax.experimental import pallas as pl
from jax.experimental.pallas import tpu as pltpu
from jax.experimental.pallas import tpu_sc as plsc

NUM_CODES = 8192
DIM = 256
NUM_TOKENS = 8192
TM = 256
TN = 2048
N_TOK_TILES = NUM_TOKENS // TM
N_CODE_TILES = NUM_CODES // TN


def _argmin_body(x_ref, e_ref, out_ref, dbuf_ref, esq_ref, xsq_ref):
    s = pl.program_id(0)

    @pl.when(s == 0)
    def _():
        e0 = e_ref[...]
        esq_ref[...] = jnp.broadcast_to(
            jnp.sum(e0 * e0, axis=1)[None, :], (8, NUM_CODES))

    def step_body(wbuf, rbuf):
        x = x_ref[...]
        xsq = jnp.sum(x * x, axis=1, keepdims=True)
        dot2 = lax.dot_general(x * jnp.float32(-2.0), e_ref[...],
                               (((1,), (1,)), ((), ())),
                               preferred_element_type=jnp.float32)
        dbuf_ref[wbuf] = dot2
        xsq_ref[wbuf] = xsq

        dot3 = dbuf_ref[rbuf].reshape(TM // 8, 8, NUM_CODES)
        xsq3 = xsq_ref[rbuf].reshape(TM // 8, 8, 1)
        dist = (xsq3 + dot3) + esq_ref[...][None]
        m = jnp.min(dist, axis=2, keepdims=True)
        iota = lax.broadcasted_iota(
            jnp.int32, (8, NUM_CODES), 1).astype(jnp.float32)
        cand = jnp.where(dist == m, iota[None], jnp.float32(1e9))
        idx = jnp.min(cand, axis=2, keepdims=True)
        out_ref[0, :, :] = idx.reshape(TM, 1).astype(jnp.int32)

    @pl.when(s % 2 == 0)
    def _():
        step_body(0, 1)

    @pl.when(s % 2 == 1)
    def _():
        step_body(1, 0)


def _argmin_call(flat, emb):
    out = pl.pallas_call(
        _argmin_body,
        grid=(N_TOK_TILES + 1,),
        in_specs=[
            pl.BlockSpec((TM, DIM),
                         lambda s: (jnp.minimum(s, N_TOK_TILES - 1), 0)),
            pl.BlockSpec((NUM_CODES, DIM), lambda s: (0, 0)),
        ],
        out_specs=pl.BlockSpec((1, TM, 1),
                               lambda s: (jnp.maximum(s, 1) - 1, 0, 0)),
        out_shape=jax.ShapeDtypeStruct((N_TOK_TILES, TM, 1), jnp.int32),
        scratch_shapes=[
            pltpu.VMEM((2, TM, NUM_CODES), jnp.float32),
            pltpu.VMEM((8, NUM_CODES), jnp.float32),
            pltpu.VMEM((2, TM, 1), jnp.float32),
        ],
    )(flat, emb)
    return out.reshape(NUM_TOKENS)


_NC = 2
_NS = 16
_NW = _NC * _NS
_CHUNK = 128
_ROWS = NUM_TOKENS // _CHUNK
_RPW = _ROWS // _NW


def _sc_gather_body(table_hbm, idx_hbm, out_hbm, idx_v, rows_v, sem):
    wid = lax.axis_index("s") * _NC + lax.axis_index("c")
    r0 = wid * _RPW
    pltpu.sync_copy(idx_hbm.at[pl.ds(r0, _RPW)], idx_v)
    cps = [
        pltpu.async_copy(table_hbm.at[idx_v.at[r]], rows_v.at[r], sem)
        for r in range(_RPW)
    ]
    for cp in cps:
        cp.wait()
    pltpu.sync_copy(rows_v, out_hbm.at[pl.ds(r0, _RPW)])


@functools.cache
def _sc_gather():
    return pl.kernel(
        _sc_gather_body,
        mesh=plsc.VectorSubcoreMesh(core_axis_name="c", subcore_axis_name="s"),
        out_type=jax.ShapeDtypeStruct((_ROWS, _CHUNK, DIM), jnp.float32),
        scratch_types=[
            pltpu.VMEM((_RPW, _CHUNK), jnp.int32),
            pltpu.VMEM((_RPW, _CHUNK, DIM), jnp.float32),
            pltpu.SemaphoreType.DMA,
        ],
    )


FT = 1024
N_FIN = NUM_TOKENS // FT


def _finalize_body(x_ref, q_ref, qst_ref, loss_ref, acc_ref):
    i = pl.program_id(0)
    x = x_ref[...]
    q = q_ref[...]
    d = q - x
    qst_ref[...] = x + d
    s = jnp.sum(d * d)

    @pl.when(i == 0)
    def _():
        acc_ref[0, 0] = s

    @pl.when(i > 0)
    def _():
        acc_ref[0, 0] = acc_ref[0, 0] + s

    @pl.when(i == pl.num_programs(0) - 1)
    def _():
        m = acc_ref[0, 0] / jnp.float32(NUM_TOKENS * DIM)
        loss_ref[0, 0] = m + 0.25 * m


def _finalize_call(flat, q):
    return pl.pallas_call(
        _finalize_body,
        grid=(N_FIN,),
        in_specs=[
            pl.BlockSpec((FT, DIM), lambda i: (i, 0)),
            pl.BlockSpec((FT, DIM), lambda i: (i, 0)),
        ],
        out_specs=[
            pl.BlockSpec((FT, DIM), lambda i: (i, 0)),
            pl.BlockSpec(memory_space=pltpu.SMEM),
        ],
        out_shape=[
            jax.ShapeDtypeStruct((NUM_TOKENS, DIM), jnp.float32),
            jax.ShapeDtypeStruct((1, 1), jnp.float32),
        ],
        scratch_shapes=[pltpu.SMEM((1, 1), jnp.float32)],
    )(flat, q)


def kernel(inputs, embedding):
    B, C, H, W = inputs.shape
    flat = jnp.transpose(inputs, (0, 2, 3, 1)).reshape(NUM_TOKENS, DIM)
    idx = _argmin_call(flat, embedding)
    q = _sc_gather()(embedding, idx.reshape(_ROWS, _CHUNK)).reshape(NUM_TOKENS, DIM)
    qst_flat, loss = _finalize_call(flat, q)
    qst = jnp.transpose(qst_flat.reshape(B, H, W, C), (0, 3, 1, 2))
    return qst, loss[0, 0], idx.reshape(B, H, W)
